# baseline (speedup 1.0000x reference)
import jax
import jax.numpy as jnp
from jax import lax
from jax.experimental import pallas as pl
from jax.experimental.pallas import tpu as pltpu
from jax.experimental.pallas import tpu_sc as plsc

B, S, H = 4, 2048, 1024
EPS = 1e-05

NC, NS = 2, 16
NW = NC * NS
NTOK = B * S
SPW = S // NW
T = 16
NSC = SPW // T
NCHUNK = NSC * B
HV = H // 16


def _lane_shuffle(v, idx):
    dnums = lax.GatherDimensionNumbers(
        offset_dims=(), collapsed_slice_dims=(0,), start_index_map=(0,))
    return lax.gather(v, idx.reshape(16, 1), dnums, (1,),
                      mode=lax.GatherScatterMode.PROMISE_IN_BOUNDS)


def _allsum(v):
    for k in (8, 4, 2, 1):
        idx = jnp.bitwise_xor(lax.iota(jnp.int32, 16), k)
        v = v + _lane_shuffle(v, idx)
    return v


def _pass1(wb, pb, ob, stat_a, stat_b):
    @plsc.parallel_loop(0, T, unroll=2)
    def _p1(t):
        zero = jnp.zeros((16,), jnp.float32)

        def j4_body(j4, accs):
            accs = list(accs)
            es = []
            for u in range(16):
                e = wb[t, pl.ds(j4 * 256 + u * 16, 16)] \
                    + pb[t, pl.ds(j4 * 256 + u * 16, 16)]
                es.append(e)
                accs[u % 4] = accs[u % 4] + e
                accs[4 + u % 4] = accs[4 + u % 4] + e * e
            for u in range(16):
                ob[t, pl.ds(j4 * 256 + u * 16, 16)] = es[u]
            return tuple(accs)

        accs = lax.fori_loop(0, HV // 16, j4_body, (zero,) * 8)
        s = (accs[0] + accs[1]) + (accs[2] + accs[3])
        q = (accs[4] + accs[5]) + (accs[6] + accs[7])
        mean = _allsum(s) * (1.0 / H)
        var = _allsum(q) * (1.0 / H) - mean * mean
        x = var + EPS
        i = lax.bitcast_convert_type(x, jnp.int32)
        i = jnp.int32(0x5F3759DF) - jnp.right_shift(i, 1)
        y = lax.bitcast_convert_type(i, jnp.float32)
        y = y * (1.5 - 0.5 * x * y * y)
        y = y * (1.5 - 0.5 * x * y * y)
        stat_a[t, :] = y
        stat_b[t, :] = -mean * y


def _pass2(ob, gv, bv, stat_a, stat_b):
    a_regs = [stat_a[t, :] for t in range(T)]
    b_regs = [stat_b[t, :] for t in range(T)]

    @plsc.parallel_loop(0, HV, unroll=2)
    def _p2(j):
        d = pl.ds(j * 16, 16)
        g = gv[d]
        be = bv[d]
        es = [ob[t, d] for t in range(T)]
        ys = [(es[t] * a_regs[t] + b_regs[t]) * g + be for t in range(T)]
        for t in range(T):
            ob[t, d] = ys[t]


def _ln_body(ids_hbm, word_hbm, pos_hbm, gamma_hbm, beta_hbm, out_hbm,
             idxall, wb0, wb1, ob0, ob1, pb0, pb1, gv, bv,
             stat_a, stat_b, isem, g0, g1, o0, o1, p0, p1):
    wid = lax.axis_index("s") * NC + lax.axis_index("c")
    sbase = wid * SPW

    WB = [wb0, wb1]
    OB = [ob0, ob1]
    GS = [g0, g1]
    OS = [o0, o1]
    PB = [pb0, pb1]
    PS = [p0, p1]

    pltpu.sync_copy(gamma_hbm, gv)
    pltpu.sync_copy(beta_hbm, bv)

    ih = [pltpu.async_copy(ids_hbm.at[pl.ds(b * S + sbase, SPW)],
                           idxall.at[pl.ds(b * SPW, SPW)], isem)
          for b in range(B)]
    for h in ih:
        h.wait()

    ph = {}
    gh = {}
    wh = {}

    def issue_pos(sc):
        ph[sc] = pltpu.async_copy(
            pos_hbm.at[pl.ds(sbase + sc * T, T)], PB[sc % 2], PS[sc % 2])

    def issue_gather(c):
        sc, b = divmod(c, B)
        idx = idxall.at[pl.ds(b * SPW + sc * T, T)]
        gh[c] = pltpu.async_copy(word_hbm.at[idx], WB[c % 2], GS[c % 2])

    issue_pos(0)
    issue_gather(0)
    issue_gather(1)

    for c in range(NCHUNK):
        sc, b = divmod(c, B)
        if b == 0 and c + B < NCHUNK:
            issue_pos(sc + 1)
        if c >= 2:
            wh[c - 2].wait()
        gh[c].wait()
        if b == 0:
            ph[sc].wait()
        _pass1(WB[c % 2], PB[sc % 2], OB[c % 2], stat_a, stat_b)
        if c + 2 < NCHUNK:
            issue_gather(c + 2)
        _pass2(OB[c % 2], gv, bv, stat_a, stat_b)
        wh[c] = pltpu.async_copy(
            OB[c % 2], out_hbm.at[pl.ds(b * S + sbase + sc * T, T)], OS[c % 2])

    wh[NCHUNK - 2].wait()
    wh[NCHUNK - 1].wait()


@jax.jit
def _ernie_embed(ids_flat, word_table, pos_table, gamma, beta):
    mesh = plsc.VectorSubcoreMesh(core_axis_name="c", subcore_axis_name="s")
    k = pl.kernel(
        _ln_body,
        out_type=jax.ShapeDtypeStruct((NTOK, H), jnp.float32),
        mesh=mesh,
        scratch_types=[
            pltpu.VMEM((B * SPW,), jnp.int32),
            pltpu.VMEM((T, H), jnp.float32),
            pltpu.VMEM((T, H), jnp.float32),
            pltpu.VMEM((T, H), jnp.float32),
            pltpu.VMEM((T, H), jnp.float32),
            pltpu.VMEM((T, H), jnp.float32),
            pltpu.VMEM((T, H), jnp.float32),
            pltpu.VMEM((H,), jnp.float32),
            pltpu.VMEM((H,), jnp.float32),
            pltpu.VMEM((T, 16), jnp.float32),
            pltpu.VMEM((T, 16), jnp.float32),
            pltpu.SemaphoreType.DMA,
            pltpu.SemaphoreType.DMA,
            pltpu.SemaphoreType.DMA,
            pltpu.SemaphoreType.DMA,
            pltpu.SemaphoreType.DMA,
            pltpu.SemaphoreType.DMA,
            pltpu.SemaphoreType.DMA,
        ],
    )
    return k(ids_flat, word_table, pos_table, gamma, beta)


def kernel(input_ids, word_table, pos_table, gamma, beta):
    pos_used = lax.slice_in_dim(pos_table, 2, 2 + S, axis=0)
    out = _ernie_embed(input_ids.reshape(-1), word_table, pos_used, gamma, beta)
    return out.reshape(B, S, H)


# speedup vs baseline: 1.1075x; 1.1075x over previous
import jax
import jax.numpy as jnp
from jax import lax
from jax.experimental import pallas as pl
from jax.experimental.pallas import tpu as pltpu
from jax.experimental.pallas import tpu_sc as plsc

B, S, H = 4, 2048, 1024
EPS = 1e-05

NC, NS = 2, 16
NW = NC * NS
NTOK = B * S
SPW = S // NW
T = 16
NSC = SPW // T
NCHUNK = NSC * B
HV = H // 16


def _lane_shuffle(v, idx):
    dnums = lax.GatherDimensionNumbers(
        offset_dims=(), collapsed_slice_dims=(0,), start_index_map=(0,))
    return lax.gather(v, idx.reshape(16, 1), dnums, (1,),
                      mode=lax.GatherScatterMode.PROMISE_IN_BOUNDS)


def _allsum(v):
    for k in (8, 4, 2, 1):
        idx = jnp.bitwise_xor(lax.iota(jnp.int32, 16), k)
        v = v + _lane_shuffle(v, idx)
    return v


def _pass1(wb, pb, ob, stat_a, stat_b):
    @plsc.parallel_loop(0, T, unroll=2)
    def _p1(t):
        zero = jnp.zeros((16,), jnp.float32)

        def j4_body(j4, accs):
            accs = list(accs)
            es = []
            for u in range(16):
                e = wb[t, pl.ds(j4 * 256 + u * 16, 16)] \
                    + pb[t, pl.ds(j4 * 256 + u * 16, 16)]
                es.append(e)
                accs[u % 4] = accs[u % 4] + e
                accs[4 + u % 4] = accs[4 + u % 4] + e * e
            for u in range(16):
                ob[t, pl.ds(j4 * 256 + u * 16, 16)] = es[u]
            return tuple(accs)

        accs = lax.fori_loop(0, HV // 16, j4_body, (zero,) * 8)
        s = (accs[0] + accs[1]) + (accs[2] + accs[3])
        q = (accs[4] + accs[5]) + (accs[6] + accs[7])
        mean = _allsum(s) * (1.0 / H)
        var = _allsum(q) * (1.0 / H) - mean * mean
        x = var + EPS
        i = lax.bitcast_convert_type(x, jnp.int32)
        i = jnp.int32(0x5F3759DF) - jnp.right_shift(i, 1)
        y = lax.bitcast_convert_type(i, jnp.float32)
        y = y * (1.5 - 0.5 * x * y * y)
        y = y * (1.5 - 0.5 * x * y * y)
        stat_a[t, :] = y
        stat_b[t, :] = -mean * y


def _pass2(ob, gv, bv, stat_a, stat_b):
    a_regs = [stat_a[t, :] for t in range(T)]
    b_regs = [stat_b[t, :] for t in range(T)]

    @plsc.parallel_loop(0, HV)
    def _p2(j):
        d = pl.ds(j * 16, 16)
        g = gv[d]
        be = bv[d]
        es = [ob[t, d] for t in range(T)]
        ys = [(es[t] * a_regs[t] + b_regs[t]) * g + be for t in range(T)]
        for t in range(T):
            ob[t, d] = ys[t]


def _ln_body(ids_hbm, word_hbm, pos_hbm, gamma_hbm, beta_hbm, out_hbm,
             idxall, wb0, wb1, ob0, ob1, pb0, pb1, gv, bv,
             stat_a, stat_b, isem, g0, g1, o0, o1, p0, p1):
    wid = lax.axis_index("s") * NC + lax.axis_index("c")
    sbase = wid * SPW

    WB = [wb0, wb1]
    OB = [ob0, ob1]
    GS = [g0, g1]
    OS = [o0, o1]
    PB = [pb0, pb1]
    PS = [p0, p1]

    pltpu.sync_copy(gamma_hbm, gv)
    pltpu.sync_copy(beta_hbm, bv)

    ih = [pltpu.async_copy(ids_hbm.at[pl.ds(b * S + sbase, SPW)],
                           idxall.at[pl.ds(b * SPW, SPW)], isem)
          for b in range(B)]
    for h in ih:
        h.wait()

    ph = {}
    gh = {}
    wh = {}

    def issue_pos(sc):
        ph[sc] = pltpu.async_copy(
            pos_hbm.at[pl.ds(sbase + sc * T, T)], PB[sc % 2], PS[sc % 2])

    def issue_gather(c):
        sc, b = divmod(c, B)
        idx = idxall.at[pl.ds(b * SPW + sc * T, T)]
        gh[c] = pltpu.async_copy(word_hbm.at[idx], WB[c % 2], GS[c % 2])

    issue_pos(0)
    issue_gather(0)
    issue_gather(1)

    for c in range(NCHUNK):
        sc, b = divmod(c, B)
        if b == 0 and c + B < NCHUNK:
            issue_pos(sc + 1)
        if c >= 2:
            wh[c - 2].wait()
        gh[c].wait()
        if b == 0:
            ph[sc].wait()
        _pass1(WB[c % 2], PB[sc % 2], OB[c % 2], stat_a, stat_b)
        if c + 2 < NCHUNK:
            issue_gather(c + 2)
        _pass2(OB[c % 2], gv, bv, stat_a, stat_b)
        wh[c] = pltpu.async_copy(
            OB[c % 2], out_hbm.at[pl.ds(b * S + sbase + sc * T, T)], OS[c % 2])

    wh[NCHUNK - 2].wait()
    wh[NCHUNK - 1].wait()


@jax.jit
def _ernie_embed(ids_flat, word_table, pos_table, gamma, beta):
    mesh = plsc.VectorSubcoreMesh(core_axis_name="c", subcore_axis_name="s")
    k = pl.kernel(
        _ln_body,
        out_type=jax.ShapeDtypeStruct((NTOK, H), jnp.float32),
        mesh=mesh,
        scratch_types=[
            pltpu.VMEM((B * SPW,), jnp.int32),
            pltpu.VMEM((T, H), jnp.float32),
            pltpu.VMEM((T, H), jnp.float32),
            pltpu.VMEM((T, H), jnp.float32),
            pltpu.VMEM((T, H), jnp.float32),
            pltpu.VMEM((T, H), jnp.float32),
            pltpu.VMEM((T, H), jnp.float32),
            pltpu.VMEM((H,), jnp.float32),
            pltpu.VMEM((H,), jnp.float32),
            pltpu.VMEM((T, 16), jnp.float32),
            pltpu.VMEM((T, 16), jnp.float32),
            pltpu.SemaphoreType.DMA,
            pltpu.SemaphoreType.DMA,
            pltpu.SemaphoreType.DMA,
            pltpu.SemaphoreType.DMA,
            pltpu.SemaphoreType.DMA,
            pltpu.SemaphoreType.DMA,
            pltpu.SemaphoreType.DMA,
        ],
    )
    return k(ids_flat, word_table, pos_table, gamma, beta)


def kernel(input_ids, word_table, pos_table, gamma, beta):
    pos_used = lax.slice_in_dim(pos_table, 2, 2 + S, axis=0)
    out = _ernie_embed(input_ids.reshape(-1), word_table, pos_used, gamma, beta)
    return out.reshape(B, S, H)
